# baseline (speedup 1.0000x reference)
import functools

import jax
import jax.numpy as jnp
from jax import lax
from jax.experimental import pallas as pl
from jax.experimental.pallas import tpu as pltpu
from jax.experimental.pallas import tpu_sc as plsc

ITEMS = 100000
D = 32
B = 8
NC = 2
NS = 16
NW = NC * NS
PER_W = ITEMS // NW
CH = 160
NK = 20

_mesh = plsc.VectorSubcoreMesh(core_axis_name="c", subcore_axis_name="s")


@functools.partial(
    pl.kernel,
    mesh=_mesh,
    compiler_params=pltpu.CompilerParams(use_tc_tiling_on_sc=False),
    out_type=jax.ShapeDtypeStruct((B, ITEMS, D), jnp.float32),
    scratch_types=[
        pltpu.VMEM((CH, D), jnp.float32),
        pltpu.VMEM((CH + 128,), jnp.float32),
        pltpu.VMEM((CH, D), jnp.float32),
        pltpu.VMEM((CH, D), jnp.float32),
        pltpu.VMEM((CH, D), jnp.float32),
        pltpu.VMEM((CH, D), jnp.float32),
        pltpu.SemaphoreType.DMA,
        pltpu.SemaphoreType.DMA,
        pltpu.SemaphoreType.DMA,
        pltpu.SemaphoreType.DMA,
    ],
)
def _sc_blend(nodes_hbm, emb_hbm, alpha_hbm, out_hbm,
              e_v, a_v, x0_v, x1_v, y0_v, y1_v, sx0, sx1, sy0, sy1):
    wid = lax.axis_index("s") * NC + lax.axis_index("c")
    start = (wid * PER_W) // 8 * 8
    end = ((wid + 1) * PER_W) // 8 * 8
    end = jnp.where(wid == NW - 1, ITEMS, end)

    def compute(x_v, y_v, delta):
        @plsc.parallel_loop(0, CH // 16, unroll=2)
        def body(g):
            arow = a_v[pl.ds(g * 16 + delta, 16)]
            for t in range(16):
                j = g * 16 + t
                a = jnp.full((16,), arow[t], jnp.float32)
                x0 = x_v[j, pl.ds(0, 16)]
                e0 = e_v[j, pl.ds(0, 16)]
                y_v[j, pl.ds(0, 16)] = e0 + a * (x0 - e0)
                x1 = x_v[j, pl.ds(16, 16)]
                e1 = e_v[j, pl.ds(16, 16)]
                y_v[j, pl.ds(16, 16)] = e1 + a * (x1 - e1)

    def chunk_body(k, _):
        base = jnp.minimum(start + k * CH, end - CH)
        abase = (base // 128) * 128
        delta = base - abase
        pltpu.sync_copy(emb_hbm.at[pl.ds(base, CH)], e_v)
        pltpu.sync_copy(alpha_hbm.at[pl.ds(abase, CH + 128)], a_v)
        pltpu.async_copy(nodes_hbm.at[pl.ds(base, CH)], x0_v, sx0)

        def pair_body(i, _):
            b0 = 2 * i
            pltpu.async_copy(
                nodes_hbm.at[pl.ds((b0 + 1) * ITEMS + base, CH)], x1_v, sx1)
            pltpu.make_async_copy(
                nodes_hbm.at[pl.ds(base, CH)], x0_v, sx0).wait()

            @pl.when(i > 0)
            def _():
                pltpu.make_async_copy(
                    y0_v, out_hbm.at[0, pl.ds(base, CH)], sy0).wait()

            compute(x0_v, y0_v, delta)
            pltpu.async_copy(y0_v, out_hbm.at[b0, pl.ds(base, CH)], sy0)

            pltpu.make_async_copy(
                nodes_hbm.at[pl.ds(base, CH)], x1_v, sx1).wait()

            @pl.when(i < (B // 2 - 1))
            def _():
                pltpu.async_copy(
                    nodes_hbm.at[pl.ds((b0 + 2) * ITEMS + base, CH)],
                    x0_v, sx0)

            @pl.when(i > 0)
            def _():
                pltpu.make_async_copy(
                    y1_v, out_hbm.at[0, pl.ds(base, CH)], sy1).wait()

            compute(x1_v, y1_v, delta)
            pltpu.async_copy(y1_v, out_hbm.at[b0 + 1, pl.ds(base, CH)], sy1)
            return 0

        lax.fori_loop(0, B // 2, pair_body, 0)
        pltpu.make_async_copy(y0_v, out_hbm.at[0, pl.ds(base, CH)], sy0).wait()
        pltpu.make_async_copy(y1_v, out_hbm.at[0, pl.ds(base, CH)], sy1).wait()
        return 0

    lax.fori_loop(0, NK, chunk_body, 0)


def kernel(nodes_output, embedding_table, alpha):
    return _sc_blend(nodes_output, embedding_table, alpha.reshape(ITEMS))


# speedup vs baseline: 1.2412x; 1.2412x over previous
import jax
import jax.numpy as jnp
from jax.experimental import pallas as pl

ITEMS = 100000
D = 32
B = 8
BLK = 10000


def _blend_body(x_ref, e_ref, a_ref, o_ref):
    x = x_ref[...]
    e = e_ref[...]
    a = a_ref[...]
    o_ref[0] = e + a * (x - e)


def kernel(nodes_output, embedding_table, alpha):
    nblk = ITEMS // BLK
    return pl.pallas_call(
        _blend_body,
        grid=(nblk, B),
        in_specs=[
            pl.BlockSpec((BLK, D), lambda i, b: (b * (ITEMS // BLK) + i, 0)),
            pl.BlockSpec((BLK, D), lambda i, b: (i, 0)),
            pl.BlockSpec((BLK, 1), lambda i, b: (i, 0)),
        ],
        out_specs=pl.BlockSpec((1, BLK, D), lambda i, b: (b, i, 0)),
        out_shape=jax.ShapeDtypeStruct((B, ITEMS, D), jnp.float32),
    )(nodes_output, embedding_table, alpha)
